# Initial kernel scaffold; baseline (speedup 1.0000x reference)
import jax
import jax.numpy as jnp
from jax.experimental import pallas as pl


def kernel(ref_bxyz, query_bxyz):
    raise NotImplementedError("write your pallas kernel here")


# speedup vs baseline: 3.2409x; 3.2409x over previous
import jax
import jax.numpy as jnp
from jax.experimental import pallas as pl
from jax.experimental.pallas import tpu as pltpu

_K = 32
_QT = 128
_BIG = 1e10


def _knn_body(q_ref, rt_ref, out_ref, s_ref):
    q = q_ref[...]
    rt = rt_ref[...]
    nr = rt.shape[1]
    qb = q[:, 0:1]
    rb = rt[0:1, :]
    qx0, qx1, qx2 = q[:, 1:2], q[:, 2:3], q[:, 3:4]
    rx0, rx1, rx2 = rt[1:2, :], rt[2:3, :], rt[3:4, :]
    q2 = qx0 * qx0 + qx1 * qx1 + qx2 * qx2
    r2 = rx0 * rx0 + rx1 * rx1 + rx2 * rx2

    def bf(x):
        return x.astype(jnp.bfloat16).astype(jnp.float32)

    cross = bf(qx0) * bf(rx0) + bf(qx1) * bf(rx1) + bf(qx2) * bf(rx2)
    d2 = q2 + r2 - 2.0 * cross
    s_ref[...] = jnp.where(qb == rb, d2, _BIG)

    iota = jax.lax.broadcasted_iota(jnp.int32, (q.shape[0], nr), 1)
    for k in range(_K):
        s = s_ref[...]
        m = jnp.min(s, axis=1, keepdims=True)
        idx = jnp.min(jnp.where(s == m, iota, nr), axis=1)
        out_ref[:, k : k + 1] = idx[:, None]
        s_ref[...] = jnp.where(iota == idx[:, None], float("inf"), s)


def _knn_topk(query_s, ref_t, *, interpret=False):
    nq = query_s.shape[0]
    nr = ref_t.shape[1]
    qt = min(_QT, nq)
    grid = (nq // qt,)
    return pl.pallas_call(
        _knn_body,
        grid=grid,
        in_specs=[
            pl.BlockSpec((qt, 4), lambda i: (i, 0)),
            pl.BlockSpec((4, nr), lambda i: (0, 0)),
        ],
        out_specs=pl.BlockSpec((qt, _K), lambda i: (i, 0)),
        out_shape=jax.ShapeDtypeStruct((nq, _K), jnp.int32),
        scratch_shapes=[pltpu.VMEM((qt, nr), jnp.float32)],
        interpret=interpret,
    )(query_s, ref_t)


def kernel(ref_bxyz, query_bxyz):
    nq = query_bxyz.shape[0]
    rb = ref_bxyz[:, 0].astype(jnp.int32)
    qb = query_bxyz[:, 0].astype(jnp.int32)
    order_r = jnp.argsort(rb)
    order_q = jnp.argsort(qb)
    ref_t = ref_bxyz[order_r].T
    query_s = query_bxyz[order_q]
    idx = _knn_topk(query_s, ref_t)
    edge0 = order_r.astype(jnp.int64)[idx]
    edge1 = jnp.broadcast_to(order_q.astype(jnp.int64)[:, None], (nq, _K))
    return jnp.stack([edge0.reshape(-1), edge1.reshape(-1)], axis=0)


# speedup vs baseline: 4.0660x; 1.2546x over previous
import functools

import jax
import jax.numpy as jnp
from jax.experimental import pallas as pl
from jax.experimental.pallas import tpu as pltpu

_K = 32
_QT = 128
_C = 1024
_BIG = 1e10


def _knn_body(off_ref, q_ref, r3_ref, out_ref, s_ref, *, nr, nb):
    cw = r3_ref.shape[2]
    q = q_ref[...]
    qt = q.shape[0]
    qb = q[:, 0:1]
    qx0, qx1, qx2 = q[:, 1:2], q[:, 2:3], q[:, 3:4]
    q2 = qx0 * qx0 + qx1 * qx1 + qx2 * qx2

    def bf(x):
        return x.astype(jnp.bfloat16).astype(jnp.float32)

    qb0 = bf(qx0)
    qb1 = bf(qx1)
    qb2 = bf(qx2)

    b_lo = q_ref[0, 0].astype(jnp.int32)
    b_hi = q_ref[qt - 1, 0].astype(jnp.int32)
    start = off_ref[b_lo]
    end = off_ref[b_hi + 1]
    c0 = start // cw
    c1 = (end + cw - 1) // cw

    lane = jax.lax.broadcasted_iota(jnp.int32, (qt, cw), 1)

    def compute_chunk(c, _):
        rt = r3_ref[c]
        rb = rt[0:1, :]
        rx0, rx1, rx2 = rt[1:2, :], rt[2:3, :], rt[3:4, :]
        r2 = rx0 * rx0 + rx1 * rx1 + rx2 * rx2
        cross = qb0 * bf(rx0) + qb1 * bf(rx1) + qb2 * bf(rx2)
        d2 = q2 + r2 - 2.0 * cross
        s_ref[c] = jnp.where(qb == rb, d2, _BIG)
        return 0

    jax.lax.fori_loop(c0, c1, compute_chunk, 0)

    inf = float("inf")
    idx = jnp.full((qt, 1), nr, jnp.int32)
    for k in range(_K):
        def min_chunk(c, carry):
            m, prev = carry
            gidx = c * cw + lane
            s = jnp.where(gidx == prev, inf, s_ref[c])
            s_ref[c] = s
            return jnp.minimum(m, jnp.min(s, axis=1, keepdims=True)), prev

        m, _ = jax.lax.fori_loop(
            c0, c1, min_chunk, (jnp.full((qt, 1), inf, jnp.float32), idx))

        def arg_chunk(c, best):
            gidx = c * cw + lane
            cand = jnp.where(s_ref[c] == m, gidx, nr)
            return jnp.minimum(best, jnp.min(cand, axis=1, keepdims=True))

        idx = jax.lax.fori_loop(c0, c1, arg_chunk, jnp.full((qt, 1), nr, jnp.int32))
        out_ref[:, k : k + 1] = idx


def _knn_topk(query_s, ref_3d, offsets, *, interpret=False):
    nq = query_s.shape[0]
    nc, _, c = ref_3d.shape
    nr = nc * c
    nb = offsets.shape[0] - 1
    qt = min(_QT, nq)
    grid = (nq // qt,)
    return pl.pallas_call(
        functools.partial(_knn_body, nr=nr, nb=nb),
        grid=grid,
        in_specs=[
            pl.BlockSpec(memory_space=pltpu.SMEM),
            pl.BlockSpec((qt, 4), lambda i: (i, 0)),
            pl.BlockSpec((nc, 4, c), lambda i: (0, 0, 0)),
        ],
        out_specs=pl.BlockSpec((qt, _K), lambda i: (i, 0)),
        out_shape=jax.ShapeDtypeStruct((nq, _K), jnp.int32),
        scratch_shapes=[pltpu.VMEM((nc, qt, c), jnp.float32)],
        interpret=interpret,
    )(offsets, query_s, ref_3d)


def kernel(ref_bxyz, query_bxyz):
    nq = query_bxyz.shape[0]
    nr = ref_bxyz.shape[0]
    nb = 4
    rb = ref_bxyz[:, 0].astype(jnp.int32)
    qb = query_bxyz[:, 0].astype(jnp.int32)
    order_r = jnp.argsort(rb)
    order_q = jnp.argsort(qb)
    c = min(_C, nr)
    ref_3d = ref_bxyz[order_r].T.reshape(4, nr // c, c).transpose(1, 0, 2)
    query_s = query_bxyz[order_q]
    counts = jnp.bincount(rb, length=nb)
    offsets = jnp.concatenate(
        [jnp.zeros((1,), jnp.int32), jnp.cumsum(counts).astype(jnp.int32)])
    idx = _knn_topk(query_s, ref_3d, offsets)
    edge0 = order_r.astype(jnp.int64)[idx]
    edge1 = jnp.broadcast_to(order_q.astype(jnp.int64)[:, None], (nq, _K))
    return jnp.stack([edge0.reshape(-1), edge1.reshape(-1)], axis=0)


# speedup vs baseline: 4.1496x; 1.0206x over previous
import functools

import jax
import jax.numpy as jnp
from jax.experimental import pallas as pl
from jax.experimental.pallas import tpu as pltpu

_K = 32
_QT = 32
_C = 512
_BIG = 1e10


def _knn_body(off_ref, q_ref, r3_ref, out_ref, s_ref, *, nr, nb):
    cw = r3_ref.shape[2]
    q = q_ref[...]
    qt = q.shape[0]
    qb = q[:, 0:1]
    qx0, qx1, qx2 = q[:, 1:2], q[:, 2:3], q[:, 3:4]
    q2 = qx0 * qx0 + qx1 * qx1 + qx2 * qx2

    def bf(x):
        return x.astype(jnp.bfloat16).astype(jnp.float32)

    qb0 = bf(qx0)
    qb1 = bf(qx1)
    qb2 = bf(qx2)

    b_lo = q_ref[0, 0].astype(jnp.int32)
    b_hi = q_ref[qt - 1, 0].astype(jnp.int32)
    start = off_ref[b_lo]
    end = off_ref[b_hi + 1]
    c0 = start // cw
    c1 = (end + cw - 1) // cw

    lane = jax.lax.broadcasted_iota(jnp.int32, (qt, cw), 1)

    def compute_chunk(c, _):
        rt = r3_ref[c]
        rb = rt[0:1, :]
        rx0, rx1, rx2 = rt[1:2, :], rt[2:3, :], rt[3:4, :]
        r2 = rx0 * rx0 + rx1 * rx1 + rx2 * rx2
        cross = qb0 * bf(rx0) + qb1 * bf(rx1) + qb2 * bf(rx2)
        d2 = q2 + r2 - 2.0 * cross
        s_ref[c] = jnp.where(qb == rb, d2, _BIG)
        return 0

    jax.lax.fori_loop(c0, c1, compute_chunk, 0)

    inf = float("inf")
    pi = jnp.full((qt, 1), -1, jnp.int32)
    for k in range(_K):
        def chunk_body(c, carry):
            macc, iacc, prev = carry
            gidx = c * cw + lane
            s = jnp.where(gidx == prev, inf, s_ref[c])
            s_ref[c] = s
            upd = s < macc
            macc = jnp.minimum(macc, s)
            iacc = jnp.where(upd, gidx, iacc)
            return macc, iacc, prev

        macc, iacc, _ = jax.lax.fori_loop(
            c0, c1, chunk_body,
            (jnp.full((qt, cw), inf, jnp.float32),
             jnp.full((qt, cw), nr, jnp.int32), pi))
        pv = jnp.min(macc, axis=1, keepdims=True)
        pi = jnp.min(jnp.where(macc == pv, iacc, nr), axis=1, keepdims=True)
        out_ref[:, k : k + 1] = pi


def _knn_topk(query_s, ref_3d, offsets, *, interpret=False):
    nq = query_s.shape[0]
    nc, _, c = ref_3d.shape
    nr = nc * c
    nb = offsets.shape[0] - 1
    qt = min(_QT, nq)
    grid = (nq // qt,)
    return pl.pallas_call(
        functools.partial(_knn_body, nr=nr, nb=nb),
        grid=grid,
        in_specs=[
            pl.BlockSpec(memory_space=pltpu.SMEM),
            pl.BlockSpec((qt, 4), lambda i: (i, 0)),
            pl.BlockSpec((nc, 4, c), lambda i: (0, 0, 0)),
        ],
        out_specs=pl.BlockSpec((qt, _K), lambda i: (i, 0)),
        out_shape=jax.ShapeDtypeStruct((nq, _K), jnp.int32),
        scratch_shapes=[pltpu.VMEM((nc, qt, c), jnp.float32)],
        interpret=interpret,
    )(offsets, query_s, ref_3d)


def kernel(ref_bxyz, query_bxyz):
    nq = query_bxyz.shape[0]
    nr = ref_bxyz.shape[0]
    nb = 4
    rb = ref_bxyz[:, 0].astype(jnp.int32)
    qb = query_bxyz[:, 0].astype(jnp.int32)
    order_r = jnp.argsort(rb)
    order_q = jnp.argsort(qb)
    c = min(_C, nr)
    ref_3d = ref_bxyz[order_r].T.reshape(4, nr // c, c).transpose(1, 0, 2)
    query_s = query_bxyz[order_q]
    counts = jnp.bincount(rb, length=nb)
    offsets = jnp.concatenate(
        [jnp.zeros((1,), jnp.int32), jnp.cumsum(counts).astype(jnp.int32)])
    idx = _knn_topk(query_s, ref_3d, offsets)
    edge0 = order_r.astype(jnp.int64)[idx]
    edge1 = jnp.broadcast_to(order_q.astype(jnp.int64)[:, None], (nq, _K))
    return jnp.stack([edge0.reshape(-1), edge1.reshape(-1)], axis=0)


# speedup vs baseline: 5.5629x; 1.3406x over previous
import functools

import jax
import jax.numpy as jnp
from jax.experimental import pallas as pl
from jax.experimental.pallas import tpu as pltpu

_K = 32
_QT = 128
_C = 512
_BIG = 1e10


def _knn_body(off_ref, q_ref, r3_ref, out_ref, s_ref, m_ref, i_ref, *, nr, nb):
    cw = r3_ref.shape[2]
    q = q_ref[...]
    qt = q.shape[0]
    qb = q[:, 0:1]
    qx0, qx1, qx2 = q[:, 1:2], q[:, 2:3], q[:, 3:4]
    q2 = q[:, 4:5]

    def bf(x):
        return x.astype(jnp.bfloat16).astype(jnp.float32)

    qb0 = bf(qx0)
    qb1 = bf(qx1)
    qb2 = bf(qx2)

    b_lo = q_ref[0, 0].astype(jnp.int32)
    b_hi = q_ref[qt - 1, 0].astype(jnp.int32)
    start = off_ref[b_lo]
    end = off_ref[b_hi + 1]
    c0 = start // cw
    c1 = (end + cw - 1) // cw

    lane = jax.lax.broadcasted_iota(jnp.int32, (qt, cw), 1)

    def compute_chunk(c, _):
        rt = r3_ref[c]
        rb = rt[0:1, :]
        rx0, rx1, rx2 = rt[1:2, :], rt[2:3, :], rt[3:4, :]
        r2 = rt[4:5, :]
        cross = qb0 * bf(rx0) + qb1 * bf(rx1) + qb2 * bf(rx2)
        d2 = q2 + r2 - 2.0 * cross
        s_ref[c] = jnp.where(qb == rb, d2, _BIG)
        return 0

    jax.lax.fori_loop(c0, c1, compute_chunk, 0)

    inf = float("inf")
    pi = jnp.full((qt, 1), -1, jnp.int32)
    for k in range(_K):
        g0 = c0 * cw + lane
        s0 = jnp.where(g0 == pi, inf, s_ref[c0])
        s_ref[c0] = s0
        m_ref[...] = s0
        i_ref[...] = g0

        def chunk_body(c, prev):
            gidx = c * cw + lane
            s = jnp.where(gidx == prev, inf, s_ref[c])
            s_ref[c] = s
            macc = m_ref[...]
            upd = s < macc
            m_ref[...] = jnp.minimum(macc, s)
            i_ref[...] = jnp.where(upd, gidx, i_ref[...])
            return prev

        jax.lax.fori_loop(c0 + 1, c1, chunk_body, pi)
        macc = m_ref[...]
        pv = jnp.min(macc, axis=1, keepdims=True)
        pi = jnp.min(jnp.where(macc == pv, i_ref[...], nr), axis=1, keepdims=True)
        out_ref[:, k : k + 1] = pi


def _knn_topk(query_s, ref_3d, offsets, *, interpret=False):
    nq = query_s.shape[0]
    nc, _, c = ref_3d.shape
    nr = nc * c
    nb = offsets.shape[0] - 1
    qt = min(_QT, nq)
    grid = (nq // qt,)
    return pl.pallas_call(
        functools.partial(_knn_body, nr=nr, nb=nb),
        grid=grid,
        in_specs=[
            pl.BlockSpec(memory_space=pltpu.SMEM),
            pl.BlockSpec((qt, 5), lambda i: (i, 0)),
            pl.BlockSpec((nc, 5, c), lambda i: (0, 0, 0)),
        ],
        out_specs=pl.BlockSpec((qt, _K), lambda i: (i, 0)),
        out_shape=jax.ShapeDtypeStruct((nq, _K), jnp.int32),
        scratch_shapes=[pltpu.VMEM((nc, qt, c), jnp.float32),
                        pltpu.VMEM((qt, c), jnp.float32),
                        pltpu.VMEM((qt, c), jnp.int32)],
        interpret=interpret,
    )(offsets, query_s, ref_3d)


def kernel(ref_bxyz, query_bxyz):
    nq = query_bxyz.shape[0]
    nr = ref_bxyz.shape[0]
    nb = 4
    rb = ref_bxyz[:, 0].astype(jnp.int32)
    qb = query_bxyz[:, 0].astype(jnp.int32)
    order_r = jnp.argsort(rb)
    order_q = jnp.argsort(qb)
    c = min(_C, nr)
    ref_s = ref_bxyz[order_r]
    r2 = jnp.sum(ref_s[:, 1:4] * ref_s[:, 1:4], axis=1)
    ref_3d = (jnp.concatenate([ref_s, r2[:, None]], axis=1)
              .T.reshape(5, nr // c, c).transpose(1, 0, 2))
    q_s = query_bxyz[order_q]
    q2 = jnp.sum(q_s[:, 1:4] * q_s[:, 1:4], axis=1)
    query_s = jnp.concatenate([q_s, q2[:, None]], axis=1)
    counts = jnp.bincount(rb, length=nb)
    offsets = jnp.concatenate(
        [jnp.zeros((1,), jnp.int32), jnp.cumsum(counts).astype(jnp.int32)])
    idx = _knn_topk(query_s, ref_3d, offsets)
    edge0 = order_r.astype(jnp.int64)[idx]
    edge1 = jnp.broadcast_to(order_q.astype(jnp.int64)[:, None], (nq, _K))
    return jnp.stack([edge0.reshape(-1), edge1.reshape(-1)], axis=0)
